# baseline (speedup 1.0000x reference)
import functools
import jax
import jax.numpy as jnp
from jax import lax
from jax.experimental import pallas as pl
from jax.experimental.pallas import tpu as pltpu, tpu_sc as plsc

VOCAB_D = 1024
SEQ_LEN = 2048
N_BATCH = 16
P_LEN = 40
N_REL1 = 238
HALF_P = P_LEN // 2

C = 32
N_CHUNK = 1024 // C


def _body(tokens_hbm, wte_hbm, prompts_hbm, out_hbm,
          idx_v, buf0, buf1, tok16, idxp, pbuf,
          gs0, gs1, psem):
    p = lax.axis_index("c")
    b = lax.axis_index("s")
    base = b * SEQ_LEN + P_LEN + p * (1024 - P_LEN)

    pltpu.sync_copy(tokens_hbm.at[pl.ds(base, 1024)], idx_v)

    def start_chunk(i, buf, sem):
        pltpu.async_copy(wte_hbm.at[idx_v.at[pl.ds(i * C, C)]], buf, sem)

    def wait_chunk(i, buf, sem):
        pltpu.make_async_copy(wte_hbm.at[idx_v.at[pl.ds(i * C, C)]],
                              buf, sem).wait()

    start_chunk(0, buf0, gs0)

    pltpu.sync_copy(tokens_hbm.at[pl.ds(b * SEQ_LEN + P_LEN, 16)], tok16)
    tv = tok16[pl.ds(0, 16)]
    r = (tv[1] % N_REL1) * P_LEN
    io = lax.iota(jnp.int32, 16)

    def prompt_share(pr0, cnt):
        idxp[pl.ds(0, 16)] = jnp.minimum(io, cnt - 1) + (r + pr0)
        idxp[pl.ds(16, 16)] = jnp.minimum(io + 16, cnt - 1) + (r + pr0)
        pltpu.async_copy(prompts_hbm.at[idxp], pbuf, psem).wait()
        pltpu.sync_copy(pbuf.at[pl.ds(0, cnt)],
                        out_hbm.at[pl.ds(b * SEQ_LEN + pr0, cnt)])

    @pl.when(p == 0)
    def _():
        prompt_share(0, 24)

    @pl.when(p == 1)
    def _():
        prompt_share(24, 16)

    def loop_body(j, carry):
        start_chunk(2 * j + 1, buf1, gs1)
        wait_chunk(2 * j, buf0, gs0)
        pltpu.sync_copy(buf0, out_hbm.at[pl.ds(base + (2 * j) * C, C)])

        @pl.when(j < N_CHUNK // 2 - 1)
        def _():
            start_chunk(2 * j + 2, buf0, gs0)

        wait_chunk(2 * j + 1, buf1, gs1)
        pltpu.sync_copy(buf1, out_hbm.at[pl.ds(base + (2 * j + 1) * C, C)])
        return carry

    lax.fori_loop(0, N_CHUNK // 2, loop_body, 0)


@functools.partial(
    pl.kernel,
    out_type=jax.ShapeDtypeStruct((N_BATCH * SEQ_LEN, VOCAB_D), jnp.float32),
    mesh=plsc.VectorSubcoreMesh(core_axis_name="c", subcore_axis_name="s"),
    scratch_types=[
        pltpu.VMEM((1024,), jnp.int32),
        pltpu.VMEM((C, VOCAB_D), jnp.float32),
        pltpu.VMEM((C, VOCAB_D), jnp.float32),
        pltpu.VMEM((16,), jnp.int32),
        pltpu.VMEM((32,), jnp.int32),
        pltpu.VMEM((32, VOCAB_D), jnp.float32),
        pltpu.SemaphoreType.DMA,
        pltpu.SemaphoreType.DMA,
        pltpu.SemaphoreType.DMA,
    ],
)
def _gather_kernel(tokens_hbm, wte_hbm, prompts_hbm, out_hbm, *scratch):
    _body(tokens_hbm, wte_hbm, prompts_hbm, out_hbm, *scratch)


@jax.jit
def kernel(tokens, wte_weight, prompts):
    tokens_flat = tokens.reshape(-1)
    prompts_flat = prompts.reshape(N_REL1 * P_LEN, VOCAB_D)
    out = _gather_kernel(tokens_flat, wte_weight, prompts_flat)
    return out.reshape(N_BATCH, SEQ_LEN, VOCAB_D)
